# baseline (speedup 1.0000x reference)
import functools

import jax
import jax.numpy as jnp
from jax import lax
from jax.experimental import pallas as pl
from jax.experimental.pallas import tpu as pltpu
from jax.experimental.pallas import tpu_sc as plsc

_N = 10000
_E = 320000
_D = 128
_NW = 32
_NPW = 320
_NPAD = _NW * _NPW
_C = 192
_NBUF = 3
_VL = 16
_NV = _D // _VL


def _sc_segmax_body(dst_hbm, ids_hbm, eb_hbm, out_hbm, ids_v0, ids_v1, ids_v2, rows_v, out_local, eb_v, rsem, isem):
    ids_bufs = (ids_v0, ids_v1, ids_v2)
    cid = lax.axis_index("c")
    sid = lax.axis_index("s")
    wid = sid * 2 + cid
    n_lo = wid * _NPW

    pltpu.sync_copy(eb_hbm, eb_v)
    ebv = eb_v[pl.ds(wid, _VL)]
    e_lo = ebv[0]
    e_hi = ebv[1]

    minf = jnp.full((_VL,), -jnp.inf, jnp.float32)

    base_a = (e_lo // 8) * 8
    nrows = e_hi - base_a
    nchunks = lax.max(0, (nrows + _C - 1) // _C)

    def flush(cur_id, acc):
        off = lax.max(cur_id - n_lo, 0) * _D
        for j in range(_NV):
            out_local[pl.ds(off + j * _VL, _VL)] = acc[j]

    def chunk_base(c):
        return lax.min(base_a + c * _C, _E - _C)

    def start_fetch(c, k):
        b = chunk_base(c)
        pltpu.async_copy(dst_hbm.at[pl.ds(b, _C)], rows_v.at[k], rsem.at[k])
        pltpu.async_copy(ids_hbm.at[pl.ds(b, _C)], ids_bufs[k].at[pl.ds(0, _C)], isem.at[k])

    def wait_fetch(c, k):
        b = chunk_base(c)
        pltpu.make_async_copy(dst_hbm.at[pl.ds(b, _C)], rows_v.at[k], rsem.at[k]).wait()
        pltpu.make_async_copy(ids_hbm.at[pl.ds(b, _C)], ids_bufs[k].at[pl.ds(0, _C)], isem.at[k]).wait()

    def process_chunk(c, k, carry):
        g0 = base_a + c * _C
        b = chunk_base(c)
        s = lax.max(e_lo, g0) - b
        t = lax.min(e_hi, lax.min(g0 + _C, _E)) - b
        t1 = t - 1

        @pl.when(t > s)
        def _():
            pad = ids_bufs[k][pl.ds(t1, _VL)]
            ids_bufs[k][pl.ds(t, _VL)] = jnp.full((_VL,), pad[0], jnp.int32)

        ngroups = lax.max(0, (t - s + _VL - 1) // _VL)

        def group_body(gi, rc):
            gb = s + gi * _VL
            cur_id = rc[0]
            acc = rc[1:]
            idvec = ids_bufs[k][pl.ds(gb, _VL)]
            prev = cur_id
            for u in range(_VL):
                rid = idvec[u]
                r = lax.min(gb + u, t1)
                changed = rid != prev

                @pl.when(changed)
                def _(prev=prev, acc=acc):
                    flush(prev, acc)

                penalty = jnp.where(changed, -jnp.inf, 0.0)
                pvec = jnp.full((_VL,), penalty, jnp.float32)
                acc = tuple(
                    jnp.maximum(acc[j] + pvec, rows_v[k, r, pl.ds(j * _VL, _VL)])
                    for j in range(_NV)
                )
                prev = rid
            return (prev,) + acc

        return lax.fori_loop(0, ngroups, group_body, carry)

    init = (jnp.int32(-1),) + tuple(minf for _ in range(_NV))

    npairs = (nchunks + _NBUF - 1) // _NBUF
    ntot = npairs * _NBUF

    for w in range(_NBUF - 1):
        @pl.when(w < ntot)
        def _(w=w):
            start_fetch(w, w)

    def fill(i, carry):
        for q in range(4):
            out_local[pl.ds((i * 4 + q) * _VL, _VL)] = minf
        return carry

    lax.fori_loop(0, _NPW * _D // (_VL * 4), fill, 0)

    def pair_body(p, carry):
        out = carry
        for kk in range(_NBUF):
            c = p * _NBUF + kk
            wait_fetch(c, kk)

            @pl.when(c + _NBUF - 1 < ntot)
            def _(c=c, kk=kk):
                start_fetch(c + _NBUF - 1, (kk + _NBUF - 1) % _NBUF)

            out = process_chunk(c, kk, out)
        return out

    final = lax.fori_loop(0, npairs, pair_body, init)

    flush(final[0], final[1:])

    pltpu.sync_copy(out_local, out_hbm.at[pl.ds(n_lo * _D, _NPW * _D)])


def _sc_segmax(dst, ids, eb):
    mesh = plsc.VectorSubcoreMesh(core_axis_name="c", subcore_axis_name="s")
    fn = pl.kernel(
        _sc_segmax_body,
        out_type=jax.ShapeDtypeStruct((_NPAD * _D,), jnp.float32),
        mesh=mesh,
        scratch_types=[
            pltpu.VMEM((_C + _VL,), jnp.int32),
            pltpu.VMEM((_C + _VL,), jnp.int32),
            pltpu.VMEM((_C + _VL,), jnp.int32),
            pltpu.VMEM((_NBUF, _C, _D), jnp.float32),
            pltpu.VMEM((_NPW * _D,), jnp.float32),
            pltpu.VMEM((40,), jnp.int32),
            pltpu.SemaphoreType.DMA((_NBUF,)),
            pltpu.SemaphoreType.DMA((_NBUF,)),
        ],
    )
    return fn(dst, ids, eb)


def _tc_mlp_body(eps_ref, segmax_ref, src_ref, w_ref, b_ref, out_ref):
    scale = 1.0 + eps_ref[0]
    x = segmax_ref[...] + scale * src_ref[...]
    y = jnp.dot(x, w_ref[...], preferred_element_type=jnp.float32)
    out_ref[...] = jnp.maximum(y + b_ref[...], 0.0)


def _tc_mlp(eps, segmax, src, w, b2):
    blk = 2000
    grid = _N // blk
    return pl.pallas_call(
        _tc_mlp_body,
        grid=(grid,),
        in_specs=[
            pl.BlockSpec(memory_space=pltpu.SMEM),
            pl.BlockSpec((blk, _D), lambda i: (i, 0)),
            pl.BlockSpec((blk, _D), lambda i: (i, 0)),
            pl.BlockSpec((_D, _D), lambda i: (0, 0)),
            pl.BlockSpec((1, _D), lambda i: (0, 0)),
        ],
        out_specs=pl.BlockSpec((blk, _D), lambda i: (i, 0)),
        out_shape=jax.ShapeDtypeStruct((_N, _D), jnp.float32),
    )(eps, segmax, src, w, b2)


@jax.jit
def kernel(src, edge_weight, dst, segment_ids, eps, W, b):
    ids = segment_ids.astype(jnp.int32)
    bounds = jnp.minimum(jnp.arange(40, dtype=jnp.int32) * _NPW, _NPAD)
    eb = jnp.searchsorted(ids, bounds, method="scan_unrolled").astype(jnp.int32)
    segmax_flat = _sc_segmax(dst, ids, eb)
    segmax = segmax_flat.reshape(_NPAD, _D)
    return _tc_mlp(eps, segmax, src, W, b.reshape(1, _D))


# speedup vs baseline: 1.0177x; 1.0177x over previous
import functools

import jax
import jax.numpy as jnp
from jax import lax
from jax.experimental import pallas as pl
from jax.experimental.pallas import tpu as pltpu
from jax.experimental.pallas import tpu_sc as plsc

_N = 10000
_E = 320000
_D = 128
_NW = 32
_NPW = 320
_NPAD = _NW * _NPW
_C = 320
_VL = 16
_NV = _D // _VL


def _sc_segmax_body(dst_hbm, ids_hbm, eb_hbm, out_hbm, ids_v0, ids_v1, rows_v, out_local, eb_v, rsem, isem):
    ids_bufs = (ids_v0, ids_v1)
    cid = lax.axis_index("c")
    sid = lax.axis_index("s")
    wid = sid * 2 + cid
    n_lo = wid * _NPW

    pltpu.sync_copy(eb_hbm, eb_v)
    ebv = eb_v[pl.ds(wid, _VL)]
    e_lo = ebv[0]
    e_hi = ebv[1]

    minf = jnp.full((_VL,), -jnp.inf, jnp.float32)

    base_a = (e_lo // 8) * 8
    nrows = e_hi - base_a
    nchunks = lax.max(0, (nrows + _C - 1) // _C)

    def flush(cur_id, acc):
        off = lax.max(cur_id - n_lo, 0) * _D
        for j in range(_NV):
            out_local[pl.ds(off + j * _VL, _VL)] = acc[j]

    def chunk_base(c):
        return lax.min(base_a + c * _C, _E - _C)

    def start_fetch(c, k):
        b = chunk_base(c)
        pltpu.async_copy(dst_hbm.at[pl.ds(b, _C)], rows_v.at[k, pl.ds(0, _C)], rsem.at[k])
        pltpu.async_copy(ids_hbm.at[pl.ds(b, _C)], ids_bufs[k].at[pl.ds(0, _C)], isem.at[k])

    def wait_fetch(c, k):
        b = chunk_base(c)
        pltpu.make_async_copy(dst_hbm.at[pl.ds(b, _C)], rows_v.at[k, pl.ds(0, _C)], rsem.at[k]).wait()
        pltpu.make_async_copy(ids_hbm.at[pl.ds(b, _C)], ids_bufs[k].at[pl.ds(0, _C)], isem.at[k]).wait()

    def process_chunk(c, k, carry):
        g0 = base_a + c * _C
        b = chunk_base(c)
        s = lax.max(e_lo, g0) - b
        t = lax.min(e_hi, lax.min(g0 + _C, _E)) - b
        t1 = t - 1

        @pl.when(t > s)
        def _():
            pad = ids_bufs[k][pl.ds(t1, _VL)]
            ids_bufs[k][pl.ds(t, _VL)] = jnp.full((_VL,), pad[0], jnp.int32)

        @pl.when(jnp.logical_and(t > s, ((t - s) % _VL) != 0))
        def _():
            for q in range(_VL):
                for j in range(_NV):
                    rows_v[k, t + q, pl.ds(j * _VL, _VL)] = minf

        ngroups = lax.max(0, (t - s + _VL - 1) // _VL)

        def group_body(gi, rc):
            gb = s + gi * _VL
            cur_id = rc[0]
            acc = rc[1:]
            idvec = ids_bufs[k][pl.ds(gb, _VL)]
            prev = cur_id
            for u in range(_VL):
                rid = idvec[u]
                r = gb + u
                changed = rid != prev

                @pl.when(changed)
                def _(prev=prev, acc=acc):
                    flush(prev, acc)

                penalty = jnp.where(changed, -jnp.inf, 0.0)
                pvec = jnp.full((_VL,), penalty, jnp.float32)
                acc = tuple(
                    jnp.maximum(acc[j] + pvec, rows_v[k, r, pl.ds(j * _VL, _VL)])
                    for j in range(_NV)
                )
                prev = rid
            return (prev,) + acc

        return lax.fori_loop(0, ngroups, group_body, carry)

    init = (jnp.int32(-1),) + tuple(minf for _ in range(_NV))

    npairs = (nchunks + 1) // 2
    ntot = npairs * 2

    @pl.when(ntot > 0)
    def _():
        start_fetch(0, 0)

    def fill(i, carry):
        for q in range(4):
            out_local[pl.ds((i * 4 + q) * _VL, _VL)] = minf
        return carry

    lax.fori_loop(0, _NPW * _D // (_VL * 4), fill, 0)

    def pair_body(p, carry):
        out = carry
        for kk in range(2):
            c = p * 2 + kk
            wait_fetch(c, kk)

            @pl.when(c + 1 < ntot)
            def _(c=c, kk=kk):
                start_fetch(c + 1, 1 - kk)

            out = process_chunk(c, kk, out)
        return out

    final = lax.fori_loop(0, npairs, pair_body, init)

    flush(final[0], final[1:])

    pltpu.sync_copy(out_local, out_hbm.at[pl.ds(n_lo * _D, _NPW * _D)])


def _sc_segmax(dst, ids, eb):
    mesh = plsc.VectorSubcoreMesh(core_axis_name="c", subcore_axis_name="s")
    fn = pl.kernel(
        _sc_segmax_body,
        out_type=jax.ShapeDtypeStruct((_NPAD * _D,), jnp.float32),
        mesh=mesh,
        scratch_types=[
            pltpu.VMEM((_C + _VL,), jnp.int32),
            pltpu.VMEM((_C + _VL,), jnp.int32),
            pltpu.VMEM((2, _C + _VL, _D), jnp.float32),
            pltpu.VMEM((_NPW * _D,), jnp.float32),
            pltpu.VMEM((40,), jnp.int32),
            pltpu.SemaphoreType.DMA((2,)),
            pltpu.SemaphoreType.DMA((2,)),
        ],
    )
    return fn(dst, ids, eb)


def _tc_mlp_body(eps_ref, segmax_ref, src_ref, w_ref, b_ref, out_ref):
    scale = 1.0 + eps_ref[0]
    x = segmax_ref[...] + scale * src_ref[...]
    y = jnp.dot(x, w_ref[...], preferred_element_type=jnp.float32)
    out_ref[...] = jnp.maximum(y + b_ref[...], 0.0)


def _tc_mlp(eps, segmax, src, w, b2):
    blk = 2000
    grid = _N // blk
    return pl.pallas_call(
        _tc_mlp_body,
        grid=(grid,),
        in_specs=[
            pl.BlockSpec(memory_space=pltpu.SMEM),
            pl.BlockSpec((blk, _D), lambda i: (i, 0)),
            pl.BlockSpec((blk, _D), lambda i: (i, 0)),
            pl.BlockSpec((_D, _D), lambda i: (0, 0)),
            pl.BlockSpec((1, _D), lambda i: (0, 0)),
        ],
        out_specs=pl.BlockSpec((blk, _D), lambda i: (i, 0)),
        out_shape=jax.ShapeDtypeStruct((_N, _D), jnp.float32),
    )(eps, segmax, src, w, b2)


@jax.jit
def kernel(src, edge_weight, dst, segment_ids, eps, W, b):
    ids = segment_ids.astype(jnp.int32)
    bounds = jnp.minimum(jnp.arange(40, dtype=jnp.int32) * _NPW, _NPAD)
    eb = jnp.searchsorted(ids, bounds, method="scan_unrolled").astype(jnp.int32)
    segmax_flat = _sc_segmax(dst, ids, eb)
    segmax = segmax_flat.reshape(_NPAD, _D)
    return _tc_mlp(eps, segmax, src, W, b.reshape(1, _D))
